# baseline (speedup 1.0000x reference)
import numpy as np

import jax
import jax.numpy as jnp
from jax.experimental import pallas as pl
from jax.experimental.pallas import tpu as pltpu

TB = 128


def _build_w1_idx():
    idx = np.full((140, 256), 250, np.int32)
    for ty in range(5):
        for xa in range(28):
            k = ty * 28 + xa
            for j in range(2):
                for c in range(10):
                    for B in range(12):
                        tx = xa - (2 * B + j)
                        if 0 <= tx <= 4:
                            idx[k, j * 128 + c * 12 + B] = (ty * 5 + tx) * 10 + c
    return idx


def _build_w2_idx():
    idx = np.full((640, 256), 5000, np.int32)
    for ky in range(5):
        for ci in range(10):
            for B in range(12):
                row = ky * 128 + ci * 12 + B
                for j in range(2):
                    for co in range(20):
                        for B2 in range(4):
                            kx = B - 2 * B2 - j
                            if 0 <= kx <= 4:
                                idx[row, j * 128 + co * 4 + B2] = (
                                    co * 250 + (ky * 5 + kx) * 10 + ci)
    return idx


def _build_f1_idx():
    idx = np.full(512, 320, np.int32)
    for A2 in range(4):
        for co in range(20):
            for B2 in range(4):
                idx[A2 * 128 + co * 4 + B2] = (A2 * 4 + B2) * 20 + co
    return idx


def _build_b1_idx():
    idx = np.full(128, 10, np.int32)
    for c in range(10):
        for B in range(12):
            idx[c * 12 + B] = c
    return idx


def _build_b2_idx():
    idx = np.full(128, 20, np.int32)
    for co in range(20):
        for B2 in range(4):
            idx[co * 4 + B2] = co
    return idx


_W1_IDX = _build_w1_idx()
_W2_IDX = _build_w2_idx()
_F1_IDX = _build_f1_idx()
_B1_IDX = _build_b1_idx()
_B2_IDX = _build_b2_idx()


def _net_kernel(x_ref, w1_ref, b1_ref, w2_ref, b2_ref,
                wf1_ref, bf1_ref, wf2_ref, bf2_ref,
                out_ref, p1_ref):
    f32 = jnp.float32
    w1 = w1_ref[...]

    for A in range(12):
        o0 = jnp.dot(x_ref[:, 56 * A: 56 * A + 140], w1,
                     preferred_element_type=f32)
        o1 = jnp.dot(x_ref[:, 56 * A + 28: 56 * A + 168], w1,
                     preferred_element_type=f32)
        m = jnp.maximum(jnp.maximum(o0[:, :128], o0[:, 128:]),
                        jnp.maximum(o1[:, :128], o1[:, 128:]))
        p1_ref[:, A * 128:(A + 1) * 128] = jnp.maximum(m + b1_ref[...], 0.0)

    w2 = w2_ref[...]
    flat_parts = []
    for A2 in range(4):
        o0 = jnp.dot(p1_ref[:, 256 * A2: 256 * A2 + 640], w2,
                     preferred_element_type=f32)
        o1 = jnp.dot(p1_ref[:, 256 * A2 + 128: 256 * A2 + 768], w2,
                     preferred_element_type=f32)
        m = jnp.maximum(jnp.maximum(o0[:, :128], o0[:, 128:]),
                        jnp.maximum(o1[:, :128], o1[:, 128:]))
        flat_parts.append(jnp.maximum(m + b2_ref[...], 0.0))
    flat = jnp.concatenate(flat_parts, axis=1)

    h = jnp.maximum(jnp.dot(flat, wf1_ref[...], preferred_element_type=f32)
                    + bf1_ref[...], 0.0)
    logits = jnp.dot(h, wf2_ref[...], preferred_element_type=f32) + bf2_ref[...]
    s = logits - jnp.max(logits, axis=1, keepdims=True)
    out_ref[...] = s - jnp.log(jnp.sum(jnp.exp(s), axis=1, keepdims=True))


def kernel(x, w1t, b1, w2m, b2, wf1, bf1, wf2, bf2):
    n = x.shape[0]
    n_pad = ((n + TB - 1) // TB) * TB
    x2 = x.astype(jnp.float32).reshape(n, 28 * 28)
    if n_pad != n:
        x2 = jnp.pad(x2, ((0, n_pad - n), (0, 0)))

    f32 = jnp.float32
    zero1 = jnp.zeros((1,), f32)
    w1m = jnp.concatenate([w1t.reshape(250), zero1])[_W1_IDX]
    w2big = jnp.concatenate([w2m.reshape(5000), zero1])[_W2_IDX]
    wf1m = jnp.concatenate([wf1.T, jnp.zeros((1, 50), f32)])[_F1_IDX]
    b1l = jnp.concatenate([b1.reshape(10), zero1])[_B1_IDX][None, :]
    b2l = jnp.concatenate([b2.reshape(20), zero1])[_B2_IDX][None, :]
    bf1l = bf1.reshape(1, 50)
    bf2l = bf2.reshape(1, 10)
    wf2m = wf2.T

    out = pl.pallas_call(
        _net_kernel,
        out_shape=jax.ShapeDtypeStruct((n_pad, 10), jnp.float32),
        grid_spec=pltpu.PrefetchScalarGridSpec(
            num_scalar_prefetch=0,
            grid=(n_pad // TB,),
            in_specs=[
                pl.BlockSpec((TB, 784), lambda t: (t, 0)),
                pl.BlockSpec((140, 256), lambda t: (0, 0)),
                pl.BlockSpec((1, 128), lambda t: (0, 0)),
                pl.BlockSpec((640, 256), lambda t: (0, 0)),
                pl.BlockSpec((1, 128), lambda t: (0, 0)),
                pl.BlockSpec((512, 50), lambda t: (0, 0)),
                pl.BlockSpec((1, 50), lambda t: (0, 0)),
                pl.BlockSpec((50, 10), lambda t: (0, 0)),
                pl.BlockSpec((1, 10), lambda t: (0, 0)),
            ],
            out_specs=pl.BlockSpec((TB, 10), lambda t: (t, 0)),
            scratch_shapes=[
                pltpu.VMEM((TB, 12 * 128), jnp.float32),
            ],
        ),
        compiler_params=pltpu.CompilerParams(
            dimension_semantics=("parallel",),
            vmem_limit_bytes=64 * 1024 * 1024,
        ),
    )(x2, w1m, b1l, w2big, b2l, wf1m, bf1l, wf2m, bf2l)
    return out[:n]


# speedup vs baseline: 7.1256x; 7.1256x over previous
import numpy as np

import jax
import jax.numpy as jnp
from jax.experimental import pallas as pl
from jax.experimental.pallas import tpu as pltpu

TB = 128


def _build_s1():
    s = np.zeros((2, 12, 140, 25), np.float32)
    for ty in range(5):
        for tx in range(5):
            for j in range(2):
                for B in range(12):
                    s[j, B, ty * 28 + (2 * B + j + tx), ty * 5 + tx] = 1.0
    return s


def _build_s2():
    s = np.zeros((2, 4, 12, 5), np.float32)
    for kx in range(5):
        for j in range(2):
            for B2 in range(4):
                s[j, B2, 2 * B2 + j + kx, kx] = 1.0
    return s


_S1 = _build_s1()
_S2 = _build_s2()


def _net_kernel(x_ref, w1_ref, b1_ref, w2_ref, b2_ref,
                wf1_ref, bf1_ref, wf2_ref, bf2_ref,
                out_ref, p1_ref):
    f32 = jnp.float32
    w1 = w1_ref[...]

    for A in range(12):
        o0 = jnp.dot(x_ref[:, 56 * A: 56 * A + 140], w1,
                     preferred_element_type=f32)
        o1 = jnp.dot(x_ref[:, 56 * A + 28: 56 * A + 168], w1,
                     preferred_element_type=f32)
        m = jnp.maximum(jnp.maximum(o0[:, :128], o0[:, 128:]),
                        jnp.maximum(o1[:, :128], o1[:, 128:]))
        p1_ref[:, A * 128:(A + 1) * 128] = jnp.maximum(m + b1_ref[...], 0.0)

    w2 = w2_ref[...]
    flat_parts = []
    for A2 in range(4):
        o0 = jnp.dot(p1_ref[:, 256 * A2: 256 * A2 + 640], w2,
                     preferred_element_type=f32)
        o1 = jnp.dot(p1_ref[:, 256 * A2 + 128: 256 * A2 + 768], w2,
                     preferred_element_type=f32)
        m = jnp.maximum(jnp.maximum(o0[:, :128], o0[:, 128:]),
                        jnp.maximum(o1[:, :128], o1[:, 128:]))
        flat_parts.append(jnp.maximum(m + b2_ref[...], 0.0))
    flat = jnp.concatenate(flat_parts, axis=1)

    h = jnp.maximum(jnp.dot(flat, wf1_ref[...], preferred_element_type=f32)
                    + bf1_ref[...], 0.0)
    logits = jnp.dot(h, wf2_ref[...], preferred_element_type=f32) + bf2_ref[...]
    s = logits - jnp.max(logits, axis=1, keepdims=True)
    out_ref[...] = s - jnp.log(jnp.sum(jnp.exp(s), axis=1, keepdims=True))


def kernel(x, w1t, b1, w2m, b2, wf1, bf1, wf2, bf2):
    n = x.shape[0]
    n_pad = ((n + TB - 1) // TB) * TB
    x2 = x.astype(jnp.float32).reshape(n, 28 * 28)
    if n_pad != n:
        x2 = jnp.pad(x2, ((0, n_pad - n), (0, 0)))

    f32 = jnp.float32
    t1 = jnp.einsum('jbkt,tc->kjcb', _S1, w1t.reshape(25, 10))
    w1m = jnp.pad(t1.reshape(140, 2, 120),
                  ((0, 0), (0, 0), (0, 8))).reshape(140, 256)
    w2r = w2m.reshape(20, 5, 5, 10)
    t2 = jnp.einsum('jqbx,oyxi->yibjoq', _S2, w2r)
    w2big = jnp.pad(t2.reshape(5, 120, 2, 80),
                    ((0, 0), (0, 8), (0, 0), (0, 48))).reshape(640, 256)
    tf = wf1.reshape(50, 4, 4, 20).transpose(1, 3, 2, 0)
    wf1m = jnp.pad(tf.reshape(4, 80, 50),
                   ((0, 0), (0, 48), (0, 0))).reshape(512, 50)
    b1l = jnp.pad(jnp.repeat(b1.reshape(10), 12), (0, 8))[None, :]
    b2l = jnp.pad(jnp.repeat(b2.reshape(20), 4), (0, 48))[None, :]
    bf1l = bf1.reshape(1, 50)
    bf2l = bf2.reshape(1, 10)
    wf2m = wf2.T

    out = pl.pallas_call(
        _net_kernel,
        out_shape=jax.ShapeDtypeStruct((n_pad, 10), jnp.float32),
        grid_spec=pltpu.PrefetchScalarGridSpec(
            num_scalar_prefetch=0,
            grid=(n_pad // TB,),
            in_specs=[
                pl.BlockSpec((TB, 784), lambda t: (t, 0)),
                pl.BlockSpec((140, 256), lambda t: (0, 0)),
                pl.BlockSpec((1, 128), lambda t: (0, 0)),
                pl.BlockSpec((640, 256), lambda t: (0, 0)),
                pl.BlockSpec((1, 128), lambda t: (0, 0)),
                pl.BlockSpec((512, 50), lambda t: (0, 0)),
                pl.BlockSpec((1, 50), lambda t: (0, 0)),
                pl.BlockSpec((50, 10), lambda t: (0, 0)),
                pl.BlockSpec((1, 10), lambda t: (0, 0)),
            ],
            out_specs=pl.BlockSpec((TB, 10), lambda t: (t, 0)),
            scratch_shapes=[
                pltpu.VMEM((TB, 12 * 128), jnp.float32),
            ],
        ),
        compiler_params=pltpu.CompilerParams(
            dimension_semantics=("parallel",),
            vmem_limit_bytes=64 * 1024 * 1024,
        ),
    )(x2, w1m, b1l, w2big, b2l, wf1m, bf1l, wf2m, bf2l)
    return out[:n]


# speedup vs baseline: 10.1153x; 1.4196x over previous
import numpy as np

import jax
import jax.numpy as jnp
from jax.experimental import pallas as pl
from jax.experimental.pallas import tpu as pltpu

TB = 256


def _build_s1():
    s = np.zeros((2, 12, 140, 25), np.float32)
    for ty in range(5):
        for tx in range(5):
            for j in range(2):
                for B in range(12):
                    s[j, B, ty * 28 + (2 * B + j + tx), ty * 5 + tx] = 1.0
    return s


def _build_s2():
    s = np.zeros((2, 4, 12, 5), np.float32)
    for kx in range(5):
        for j in range(2):
            for B2 in range(4):
                s[j, B2, 2 * B2 + j + kx, kx] = 1.0
    return s


_S1 = _build_s1()
_S2 = _build_s2()


def _net_kernel(x_ref, w1_ref, b1_ref, w2_ref, b2_ref,
                wf1_ref, bf1_ref, wf2_ref, bf2_ref,
                out_ref, xf_ref, p1_ref):
    f32 = jnp.float32
    w1 = w1_ref[...]

    xf_ref[...] = x_ref[...].reshape(TB, 784)

    for A in range(12):
        o0 = jnp.dot(xf_ref[:, 56 * A: 56 * A + 140], w1,
                     preferred_element_type=f32)
        o1 = jnp.dot(xf_ref[:, 56 * A + 28: 56 * A + 168], w1,
                     preferred_element_type=f32)
        m = jnp.maximum(jnp.maximum(o0[:, :128], o0[:, 128:]),
                        jnp.maximum(o1[:, :128], o1[:, 128:]))
        p1_ref[:, A * 128:(A + 1) * 128] = jnp.maximum(m + b1_ref[...], 0.0)

    w2 = w2_ref[...]
    flat_parts = []
    for A2 in range(4):
        o0 = jnp.dot(p1_ref[:, 256 * A2: 256 * A2 + 640], w2,
                     preferred_element_type=f32)
        o1 = jnp.dot(p1_ref[:, 256 * A2 + 128: 256 * A2 + 768], w2,
                     preferred_element_type=f32)
        m = jnp.maximum(jnp.maximum(o0[:, :128], o0[:, 128:]),
                        jnp.maximum(o1[:, :128], o1[:, 128:]))
        flat_parts.append(jnp.maximum(m + b2_ref[...], 0.0))
    flat = jnp.concatenate(flat_parts, axis=1)

    h = jnp.maximum(jnp.dot(flat, wf1_ref[...], preferred_element_type=f32)
                    + bf1_ref[...], 0.0)
    logits = jnp.dot(h, wf2_ref[...], preferred_element_type=f32) + bf2_ref[...]
    s = logits - jnp.max(logits, axis=1, keepdims=True)
    out_ref[...] = s - jnp.log(jnp.sum(jnp.exp(s), axis=1, keepdims=True))


def kernel(x, w1t, b1, w2m, b2, wf1, bf1, wf2, bf2):
    n = x.shape[0]
    n_pad = ((n + TB - 1) // TB) * TB
    x2 = x.astype(jnp.float32).reshape(n, 28, 28)
    if n_pad != n:
        x2 = jnp.pad(x2, ((0, n_pad - n), (0, 0), (0, 0)))

    f32 = jnp.float32
    t1 = jnp.einsum('jbkt,tc->kjcb', _S1, w1t.reshape(25, 10))
    w1m = jnp.pad(t1.reshape(140, 2, 120),
                  ((0, 0), (0, 0), (0, 8))).reshape(140, 256)
    w2r = w2m.reshape(20, 5, 5, 10)
    t2 = jnp.einsum('jqbx,oyxi->yibjoq', _S2, w2r)
    w2big = jnp.pad(t2.reshape(5, 120, 2, 80),
                    ((0, 0), (0, 8), (0, 0), (0, 48))).reshape(640, 256)
    tf = wf1.reshape(50, 4, 4, 20).transpose(1, 3, 2, 0)
    wf1m = jnp.pad(tf.reshape(4, 80, 50),
                   ((0, 0), (0, 48), (0, 0))).reshape(512, 50)
    b1l = jnp.pad(jnp.repeat(b1.reshape(10), 12), (0, 8))[None, :]
    b2l = jnp.pad(jnp.repeat(b2.reshape(20), 4), (0, 48))[None, :]
    bf1l = bf1.reshape(1, 50)
    bf2l = bf2.reshape(1, 10)
    wf2m = wf2.T

    out = pl.pallas_call(
        _net_kernel,
        out_shape=jax.ShapeDtypeStruct((n_pad, 10), jnp.float32),
        grid_spec=pltpu.PrefetchScalarGridSpec(
            num_scalar_prefetch=0,
            grid=(n_pad // TB,),
            in_specs=[
                pl.BlockSpec((TB, 28, 28), lambda t: (t, 0, 0)),
                pl.BlockSpec((140, 256), lambda t: (0, 0)),
                pl.BlockSpec((1, 128), lambda t: (0, 0)),
                pl.BlockSpec((640, 256), lambda t: (0, 0)),
                pl.BlockSpec((1, 128), lambda t: (0, 0)),
                pl.BlockSpec((512, 50), lambda t: (0, 0)),
                pl.BlockSpec((1, 50), lambda t: (0, 0)),
                pl.BlockSpec((50, 10), lambda t: (0, 0)),
                pl.BlockSpec((1, 10), lambda t: (0, 0)),
            ],
            out_specs=pl.BlockSpec((TB, 10), lambda t: (t, 0)),
            scratch_shapes=[
                pltpu.VMEM((TB, 784), jnp.float32),
                pltpu.VMEM((TB, 12 * 128), jnp.float32),
            ],
        ),
        compiler_params=pltpu.CompilerParams(
            dimension_semantics=("parallel",),
            vmem_limit_bytes=64 * 1024 * 1024,
        ),
    )(x2, w1m, b1l, w2big, b2l, wf1m, bf1l, wf2m, bf2l)
    return out[:n]


# speedup vs baseline: 11.7481x; 1.1614x over previous
import numpy as np

import jax
import jax.numpy as jnp
from jax.experimental import pallas as pl
from jax.experimental.pallas import tpu as pltpu

TB = 256


def _build_s1():
    s = np.zeros((2, 12, 140, 25), np.float32)
    for ty in range(5):
        for tx in range(5):
            for j in range(2):
                for B in range(12):
                    s[j, B, ty * 28 + (2 * B + j + tx), ty * 5 + tx] = 1.0
    return s


def _build_s2():
    s = np.zeros((2, 4, 12, 5), np.float32)
    for kx in range(5):
        for j in range(2):
            for B2 in range(4):
                s[j, B2, 2 * B2 + j + kx, kx] = 1.0
    return s


_S1 = _build_s1()
_S2 = _build_s2()


def _net_kernel(x_ref, w1_ref, b1_ref, w2_ref, b2_ref,
                wf1_ref, bf1_ref, wf2_ref, bf2_ref,
                out_ref, xf_ref, p1_ref):
    f32 = jnp.float32
    bf16 = jnp.bfloat16
    w1 = w1_ref[...]

    xf_ref[...] = x_ref[...].reshape(TB, 784).astype(bf16)

    for A in range(12):
        o0 = jnp.dot(xf_ref[:, 56 * A: 56 * A + 140], w1,
                     preferred_element_type=f32)
        o1 = jnp.dot(xf_ref[:, 56 * A + 28: 56 * A + 168], w1,
                     preferred_element_type=f32)
        m = jnp.maximum(jnp.maximum(o0[:, :128], o0[:, 128:]),
                        jnp.maximum(o1[:, :128], o1[:, 128:]))
        p1_ref[:, A * 128:(A + 1) * 128] = jnp.maximum(
            m + b1_ref[...], 0.0).astype(bf16)

    w2 = w2_ref[...]
    flat_parts = []
    for A2 in range(4):
        o0 = jnp.dot(p1_ref[:, 256 * A2: 256 * A2 + 640], w2,
                     preferred_element_type=f32)
        o1 = jnp.dot(p1_ref[:, 256 * A2 + 128: 256 * A2 + 768], w2,
                     preferred_element_type=f32)
        m = jnp.maximum(jnp.maximum(o0[:, :128], o0[:, 128:]),
                        jnp.maximum(o1[:, :128], o1[:, 128:]))
        flat_parts.append(jnp.maximum(m + b2_ref[...], 0.0).astype(bf16))
    flat = jnp.concatenate(flat_parts, axis=1)

    h = jnp.maximum(jnp.dot(flat, wf1_ref[...], preferred_element_type=f32)
                    + bf1_ref[...], 0.0).astype(bf16)
    logits = jnp.dot(h, wf2_ref[...], preferred_element_type=f32) + bf2_ref[...]
    s = logits - jnp.max(logits, axis=1, keepdims=True)
    out_ref[...] = s - jnp.log(jnp.sum(jnp.exp(s), axis=1, keepdims=True))


def kernel(x, w1t, b1, w2m, b2, wf1, bf1, wf2, bf2):
    n = x.shape[0]
    n_pad = ((n + TB - 1) // TB) * TB
    x2 = x.astype(jnp.float32).reshape(n, 28, 28)
    if n_pad != n:
        x2 = jnp.pad(x2, ((0, n_pad - n), (0, 0), (0, 0)))

    f32 = jnp.float32
    t1 = jnp.einsum('jbkt,tc->kjcb', _S1, w1t.reshape(25, 10))
    w1m = jnp.pad(t1.reshape(140, 2, 120),
                  ((0, 0), (0, 0), (0, 8))).reshape(140, 256)
    w2r = w2m.reshape(20, 5, 5, 10)
    t2 = jnp.einsum('jqbx,oyxi->yibjoq', _S2, w2r)
    w2big = jnp.pad(t2.reshape(5, 120, 2, 80),
                    ((0, 0), (0, 8), (0, 0), (0, 48))).reshape(640, 256)
    tf = wf1.reshape(50, 4, 4, 20).transpose(1, 3, 2, 0)
    wf1m = jnp.pad(tf.reshape(4, 80, 50),
                   ((0, 0), (0, 48), (0, 0))).reshape(512, 50)
    b1l = jnp.pad(jnp.repeat(b1.reshape(10), 12), (0, 8))[None, :]
    b2l = jnp.pad(jnp.repeat(b2.reshape(20), 4), (0, 48))[None, :]
    bf1l = bf1.reshape(1, 50)
    bf2l = bf2.reshape(1, 10)
    bf16 = jnp.bfloat16
    w1m = w1m.astype(bf16)
    w2big = w2big.astype(bf16)
    wf1m = wf1m.astype(bf16)
    wf2m = wf2.T.astype(bf16)

    out = pl.pallas_call(
        _net_kernel,
        out_shape=jax.ShapeDtypeStruct((n_pad, 10), jnp.float32),
        grid_spec=pltpu.PrefetchScalarGridSpec(
            num_scalar_prefetch=0,
            grid=(n_pad // TB,),
            in_specs=[
                pl.BlockSpec((TB, 28, 28), lambda t: (t, 0, 0)),
                pl.BlockSpec((140, 256), lambda t: (0, 0)),
                pl.BlockSpec((1, 128), lambda t: (0, 0)),
                pl.BlockSpec((640, 256), lambda t: (0, 0)),
                pl.BlockSpec((1, 128), lambda t: (0, 0)),
                pl.BlockSpec((512, 50), lambda t: (0, 0)),
                pl.BlockSpec((1, 50), lambda t: (0, 0)),
                pl.BlockSpec((50, 10), lambda t: (0, 0)),
                pl.BlockSpec((1, 10), lambda t: (0, 0)),
            ],
            out_specs=pl.BlockSpec((TB, 10), lambda t: (t, 0)),
            scratch_shapes=[
                pltpu.VMEM((TB, 784), jnp.bfloat16),
                pltpu.VMEM((TB, 12 * 128), jnp.bfloat16),
            ],
        ),
        compiler_params=pltpu.CompilerParams(
            dimension_semantics=("parallel",),
            vmem_limit_bytes=64 * 1024 * 1024,
        ),
    )(x2, w1m, b1l, w2big, b2l, wf1m, bf1l, wf2m, bf2l)
    return out[:n]


# speedup vs baseline: 11.7665x; 1.0016x over previous
import numpy as np

import jax
import jax.numpy as jnp
from jax.experimental import pallas as pl
from jax.experimental.pallas import tpu as pltpu

TB = 256


def _build_s1():
    s = np.zeros((140, 2, 12, 25), np.float32)
    for ty in range(5):
        for tx in range(5):
            for j in range(2):
                for B in range(12):
                    s[ty * 28 + (2 * B + j + tx), j, B, ty * 5 + tx] = 1.0
    return s


def _build_s2():
    s = np.zeros((2, 4, 12, 5), np.float32)
    for kx in range(5):
        for j in range(2):
            for B2 in range(4):
                s[j, B2, 2 * B2 + j + kx, kx] = 1.0
    return s


_S1 = _build_s1()
_S2 = _build_s2()


def _net_kernel(x_ref, w1_ref, b1_ref, w2_ref, b2_ref,
                wf1_ref, bf1_ref, wf2_ref, bf2_ref,
                out_ref, xf_ref, p1_ref):
    f32 = jnp.float32
    bf16 = jnp.bfloat16
    w1 = w1_ref[...]

    xf_ref[...] = x_ref[...].reshape(TB, 784).astype(bf16)

    for A in range(12):
        o0 = jnp.dot(xf_ref[:, 56 * A: 56 * A + 140], w1,
                     preferred_element_type=f32)
        o1 = jnp.dot(xf_ref[:, 56 * A + 28: 56 * A + 168], w1,
                     preferred_element_type=f32)
        m = jnp.maximum(jnp.maximum(o0[:, :128], o0[:, 128:]),
                        jnp.maximum(o1[:, :128], o1[:, 128:]))
        p1_ref[:, A * 128:(A + 1) * 128] = jnp.maximum(
            m + b1_ref[...], 0.0).astype(bf16)

    w2 = w2_ref[...]
    flat_parts = []
    for A2 in range(4):
        o0 = jnp.dot(p1_ref[:, 256 * A2: 256 * A2 + 640], w2,
                     preferred_element_type=f32)
        o1 = jnp.dot(p1_ref[:, 256 * A2 + 128: 256 * A2 + 768], w2,
                     preferred_element_type=f32)
        m = jnp.maximum(jnp.maximum(o0[:, :128], o0[:, 128:]),
                        jnp.maximum(o1[:, :128], o1[:, 128:]))
        flat_parts.append(jnp.maximum(m + b2_ref[...], 0.0).astype(bf16))
    flat = jnp.concatenate(flat_parts, axis=1)

    h = jnp.maximum(jnp.dot(flat, wf1_ref[...], preferred_element_type=f32)
                    + bf1_ref[...], 0.0).astype(bf16)
    logits = jnp.dot(h, wf2_ref[...], preferred_element_type=f32) + bf2_ref[...]
    s = logits - jnp.max(logits, axis=1, keepdims=True)
    out_ref[...] = s - jnp.log(jnp.sum(jnp.exp(s), axis=1, keepdims=True))


def kernel(x, w1t, b1, w2m, b2, wf1, bf1, wf2, bf2):
    n = x.shape[0]
    n_pad = ((n + TB - 1) // TB) * TB
    x2 = x.astype(jnp.float32).reshape(n, 28, 28)
    if n_pad != n:
        x2 = jnp.pad(x2, ((0, n_pad - n), (0, 0), (0, 0)))

    f32 = jnp.float32
    bf16 = jnp.bfloat16
    t1 = jnp.einsum('kjbt,tc->kjbc', _S1, w1t.reshape(25, 10))
    w1m = jnp.pad(t1.reshape(140, 2, 120),
                  ((0, 0), (0, 0), (0, 8))).reshape(140, 256).astype(bf16)
    w2r = w2m.reshape(20, 5, 5, 10)
    t2 = jnp.einsum('jqbx,oyxi->ybijqo', _S2, w2r)
    w2big = jnp.pad(t2.reshape(5, 120, 2, 80),
                    ((0, 0), (0, 8), (0, 0), (0, 48))
                    ).reshape(640, 256).astype(bf16)
    wf1m = jnp.pad(wf1.T.reshape(4, 80, 50),
                   ((0, 0), (0, 48), (0, 0))).reshape(512, 50).astype(bf16)
    b1l = jnp.pad(jnp.tile(b1.reshape(10), 12), (0, 8))[None, :]
    b2l = jnp.pad(jnp.tile(b2.reshape(20), 4), (0, 48))[None, :]
    bf1l = bf1.reshape(1, 50)
    bf2l = bf2.reshape(1, 10)
    wf2m = wf2.T.astype(bf16)

    out = pl.pallas_call(
        _net_kernel,
        out_shape=jax.ShapeDtypeStruct((n_pad, 10), jnp.float32),
        grid_spec=pltpu.PrefetchScalarGridSpec(
            num_scalar_prefetch=0,
            grid=(n_pad // TB,),
            in_specs=[
                pl.BlockSpec((TB, 28, 28), lambda t: (t, 0, 0)),
                pl.BlockSpec((140, 256), lambda t: (0, 0)),
                pl.BlockSpec((1, 128), lambda t: (0, 0)),
                pl.BlockSpec((640, 256), lambda t: (0, 0)),
                pl.BlockSpec((1, 128), lambda t: (0, 0)),
                pl.BlockSpec((512, 50), lambda t: (0, 0)),
                pl.BlockSpec((1, 50), lambda t: (0, 0)),
                pl.BlockSpec((50, 10), lambda t: (0, 0)),
                pl.BlockSpec((1, 10), lambda t: (0, 0)),
            ],
            out_specs=pl.BlockSpec((TB, 10), lambda t: (t, 0)),
            scratch_shapes=[
                pltpu.VMEM((TB, 784), jnp.bfloat16),
                pltpu.VMEM((TB, 12 * 128), jnp.bfloat16),
            ],
        ),
        compiler_params=pltpu.CompilerParams(
            dimension_semantics=("parallel",),
            vmem_limit_bytes=64 * 1024 * 1024,
        ),
    )(x2, w1m, b1l, w2big, b2l, wf1m, bf1l, wf2m, bf2l)
    return out[:n]
